# baseline (speedup 1.0000x reference)
import functools

import jax
import jax.numpy as jnp
from jax import lax
from jax.experimental import pallas as pl
from jax.experimental.pallas import tpu as pltpu
from jax.experimental.pallas import tpu_sc as plsc

N_MOL = 16384
N_ATOMS = 2097152
NC = 2
NS = 16
NW = NC * NS
CH = 16384
N_CHUNK = N_ATOMS // (NW * CH)
SL = N_MOL // NS
STRIDE = 8192


@functools.partial(
    pl.kernel,
    out_type=jax.ShapeDtypeStruct((NC, N_MOL), jnp.float32),
    mesh=plsc.VectorSubcoreMesh(core_axis_name="c", subcore_axis_name="s"),
    scratch_types=[
        pltpu.VMEM((CH,), jnp.int32),
        pltpu.VMEM((CH,), jnp.int32),
        pltpu.VMEM((CH,), jnp.int32),
        pltpu.VMEM((CH,), jnp.int32),
        pltpu.VMEM((CH,), jnp.float32),
        pltpu.VMEM((CH,), jnp.float32),
        pltpu.VMEM_SHARED((N_MOL,), jnp.float32),
        pltpu.VMEM_SHARED((128,), jnp.float32),
        pltpu.SemaphoreType.DMA,
        pltpu.SemaphoreType.DMA,
        pltpu.SemaphoreType.DMA,
        pltpu.SemaphoreType.DMA,
    ],
)
def _sc_scatter(zm_hbm, aref_hbm, out_hbm,
                z0_v, z1_v, m0_v, m1_v, v0_v, v1_v, acc_sh, aref_sh,
                ld_sem0, ld_sem1, st_sem0, st_sem1):
    cid = lax.axis_index("c")
    sid = lax.axis_index("s")
    wid = sid * NC + cid

    zero16 = jnp.zeros((16,), jnp.float32)
    for j in range(SL // 16):
        v0_v[pl.ds(j * 16, 16)] = zero16
    pltpu.sync_copy(v0_v.at[pl.ds(0, SL)],
                    acc_sh.at[pl.ds(sid * SL, SL)])

    @pl.when(sid == 0)
    def _():
        pltpu.sync_copy(aref_hbm, aref_sh)

    plsc.subcore_barrier()

    base = wid * N_CHUNK
    z_bufs = [z0_v, z1_v]
    m_bufs = [m0_v, m1_v]
    v_bufs = [v0_v, v1_v]
    ld_sems = [ld_sem0, ld_sem1]
    st_sems = [st_sem0, st_sem1]
    loads = [None, None]
    scats = [None, None]
    loads[0] = (
        pltpu.async_copy(zm_hbm.at[base, 0], z_bufs[0], ld_sems[0]),
        pltpu.async_copy(zm_hbm.at[base, 1], m_bufs[0], ld_sems[0]),
    )
    for i in range(N_CHUNK):
        b = i % 2
        nb = (i + 1) % 2
        loads[b][0].wait()
        loads[b][1].wait()
        if i + 1 < N_CHUNK:
            if scats[nb] is not None:
                scats[nb].wait()
            loads[nb] = (
                pltpu.async_copy(zm_hbm.at[base + i + 1, 0],
                                 z_bufs[nb], ld_sems[nb]),
                pltpu.async_copy(zm_hbm.at[base + i + 1, 1],
                                 m_bufs[nb], ld_sems[nb]),
            )
        pltpu.sync_copy(aref_sh.at[z_bufs[b]], v_bufs[b])
        scats[b] = pltpu.async_copy(v_bufs[b], acc_sh.at[m_bufs[b]],
                                    st_sems[b], add=True)
    for s in scats:
        if s is not None:
            s.wait()

    plsc.subcore_barrier()
    pltpu.sync_copy(acc_sh.at[pl.ds(sid * SL, SL)],
                    out_hbm.at[cid, pl.ds(sid * SL, SL)])


def _combine_body(mean_ref, energy_ref, n_ref, p_ref, o_ref):
    o_ref[...] = (energy_ref[...]
                  + mean_ref[0] * n_ref[...].astype(jnp.float32)
                  - p_ref[0] - p_ref[1])


def _interleave(x):
    return x.reshape(N_ATOMS // STRIDE, STRIDE).T.reshape(N_ATOMS // CH, CH)


def kernel(energy, n_atoms, idx_m, Z, mean, atomref):
    zi = _interleave(Z.astype(jnp.int32))
    mi = _interleave(idx_m.astype(jnp.int32))
    zm = jnp.stack([zi, mi], axis=1)
    aref128 = jnp.pad(atomref.astype(jnp.float32),
                      (0, 128 - atomref.shape[0]))
    partials = _sc_scatter(zm, aref128)

    e2 = pl.pallas_call(
        _combine_body,
        out_shape=jax.ShapeDtypeStruct((128, 128), jnp.float32),
        in_specs=[
            pl.BlockSpec(memory_space=pltpu.SMEM),
            pl.BlockSpec(memory_space=pltpu.VMEM),
            pl.BlockSpec(memory_space=pltpu.VMEM),
            pl.BlockSpec(memory_space=pltpu.VMEM),
        ],
        out_specs=pl.BlockSpec(memory_space=pltpu.VMEM),
    )(mean, energy.reshape(128, 128),
      n_atoms.astype(jnp.int32).reshape(128, 128),
      partials.reshape(NC, 128, 128))
    return e2.reshape(N_MOL)


# speedup vs baseline: 1.4158x; 1.4158x over previous
import functools

import jax
import jax.numpy as jnp
from jax import lax
from jax.experimental import pallas as pl
from jax.experimental.pallas import tpu as pltpu
from jax.experimental.pallas import tpu_sc as plsc

N_MOL = 16384
N_ATOMS = 2097152
NC = 2
NS = 16
NW = NC * NS
CH = 16384
N_CHUNK = N_ATOMS // (NW * CH)
SL = N_MOL // NS
STRIDE = 8192


@functools.partial(
    pl.kernel,
    out_type=jax.ShapeDtypeStruct((NC, N_MOL), jnp.float32),
    mesh=plsc.VectorSubcoreMesh(core_axis_name="c", subcore_axis_name="s"),
    scratch_types=[
        pltpu.VMEM((CH,), jnp.int32),
        pltpu.VMEM((CH,), jnp.int32),
        pltpu.VMEM((CH,), jnp.int32),
        pltpu.VMEM((CH,), jnp.int32),
        pltpu.VMEM((CH,), jnp.float32),
        pltpu.VMEM((CH,), jnp.float32),
        pltpu.VMEM_SHARED((N_MOL,), jnp.float32),
        pltpu.VMEM_SHARED((128,), jnp.float32),
        pltpu.SemaphoreType.DMA,
        pltpu.SemaphoreType.DMA,
        pltpu.SemaphoreType.DMA,
        pltpu.SemaphoreType.DMA,
    ],
)
def _sc_scatter(z_hbm, m_hbm, aref_hbm, out_hbm,
                z0_v, z1_v, m0_v, m1_v, v0_v, v1_v, acc_sh, aref_sh,
                ld_sem0, ld_sem1, st_sem0, st_sem1):
    cid = lax.axis_index("c")
    sid = lax.axis_index("s")
    wid = sid * NC + cid

    zero16 = jnp.zeros((16,), jnp.float32)
    for j in range(SL // 16):
        v0_v[pl.ds(j * 16, 16)] = zero16
    pltpu.sync_copy(v0_v.at[pl.ds(0, SL)],
                    acc_sh.at[pl.ds(sid * SL, SL)])

    @pl.when(sid == 0)
    def _():
        pltpu.sync_copy(aref_hbm, aref_sh)

    plsc.subcore_barrier()

    base = wid * N_CHUNK
    z_bufs = [z0_v, z1_v]
    m_bufs = [m0_v, m1_v]
    v_bufs = [v0_v, v1_v]
    ld_sems = [ld_sem0, ld_sem1]
    st_sems = [st_sem0, st_sem1]
    loads = [None, None]
    scats = [None, None]
    loads[0] = (
        pltpu.async_copy(z_hbm.at[base], z_bufs[0], ld_sems[0]),
        pltpu.async_copy(m_hbm.at[base], m_bufs[0], ld_sems[0]),
    )
    for i in range(N_CHUNK):
        b = i % 2
        nb = (i + 1) % 2
        loads[b][0].wait()
        loads[b][1].wait()
        if i + 1 < N_CHUNK:
            if scats[nb] is not None:
                scats[nb].wait()
            loads[nb] = (
                pltpu.async_copy(z_hbm.at[base + i + 1],
                                 z_bufs[nb], ld_sems[nb]),
                pltpu.async_copy(m_hbm.at[base + i + 1],
                                 m_bufs[nb], ld_sems[nb]),
            )
        pltpu.sync_copy(aref_sh.at[z_bufs[b]], v_bufs[b])
        scats[b] = pltpu.async_copy(v_bufs[b], acc_sh.at[m_bufs[b]],
                                    st_sems[b], add=True)
    for s in scats:
        if s is not None:
            s.wait()

    plsc.subcore_barrier()
    pltpu.sync_copy(acc_sh.at[pl.ds(sid * SL, SL)],
                    out_hbm.at[cid, pl.ds(sid * SL, SL)])


def _combine_body(mean_ref, energy_ref, n_ref, p_ref, o_ref):
    o_ref[...] = (energy_ref[...]
                  + mean_ref[0] * n_ref[...].astype(jnp.float32)
                  - p_ref[0] - p_ref[1])


def _interleave(x):
    return x.reshape(N_ATOMS // STRIDE, STRIDE).T.reshape(N_ATOMS // CH, CH)


def kernel(energy, n_atoms, idx_m, Z, mean, atomref):
    zi = _interleave(Z.astype(jnp.int32))
    mi = _interleave(idx_m.astype(jnp.int32))
    aref128 = jnp.pad(atomref.astype(jnp.float32),
                      (0, 128 - atomref.shape[0]))
    partials = _sc_scatter(zi, mi, aref128)

    e2 = pl.pallas_call(
        _combine_body,
        out_shape=jax.ShapeDtypeStruct((128, 128), jnp.float32),
        in_specs=[
            pl.BlockSpec(memory_space=pltpu.SMEM),
            pl.BlockSpec(memory_space=pltpu.VMEM),
            pl.BlockSpec(memory_space=pltpu.VMEM),
            pl.BlockSpec(memory_space=pltpu.VMEM),
        ],
        out_specs=pl.BlockSpec(memory_space=pltpu.VMEM),
    )(mean, energy.reshape(128, 128),
      n_atoms.astype(jnp.int32).reshape(128, 128),
      partials.reshape(NC, 128, 128))
    return e2.reshape(N_MOL)


# speedup vs baseline: 1.9322x; 1.3648x over previous
import functools

import jax
import jax.numpy as jnp
from jax import lax
from jax.experimental import pallas as pl
from jax.experimental.pallas import tpu as pltpu
from jax.experimental.pallas import tpu_sc as plsc

N_MOL = 16384
N_ATOMS = 2097152
NC = 2
NS = 16
NW = NC * NS
CH = 16384
N_CHUNK = N_ATOMS // (NW * CH)
SL = N_MOL // NS
STRIDE = 8192
VPG = 8


@functools.partial(
    pl.kernel,
    out_type=jax.ShapeDtypeStruct((NC, N_MOL), jnp.float32),
    mesh=plsc.VectorSubcoreMesh(core_axis_name="c", subcore_axis_name="s"),
    compiler_params=pltpu.CompilerParams(needs_layout_passes=False),
    scratch_types=[
        pltpu.VMEM((CH,), jnp.int32),
        pltpu.VMEM((CH,), jnp.int32),
        pltpu.VMEM((CH,), jnp.int32),
        pltpu.VMEM((CH,), jnp.int32),
        pltpu.VMEM((CH,), jnp.float32),
        pltpu.VMEM((CH,), jnp.float32),
        pltpu.VMEM((128,), jnp.float32),
        pltpu.VMEM_SHARED((N_MOL,), jnp.float32),
        pltpu.SemaphoreType.DMA,
        pltpu.SemaphoreType.DMA,
        pltpu.SemaphoreType.DMA,
        pltpu.SemaphoreType.DMA,
    ],
)
def _sc_scatter(pk_hbm, aref_hbm, out_hbm,
                p0_v, p1_v, m0_v, m1_v, v0_v, v1_v, tab_v, acc_sh,
                ld_sem0, ld_sem1, st_sem0, st_sem1):
    cid = lax.axis_index("c")
    sid = lax.axis_index("s")
    wid = sid * NC + cid

    pltpu.sync_copy(aref_hbm, tab_v)

    zero16 = jnp.zeros((16,), jnp.float32)
    for j in range(SL // 16):
        v0_v[pl.ds(j * 16, 16)] = zero16
    pltpu.sync_copy(v0_v.at[pl.ds(0, SL)],
                    acc_sh.at[pl.ds(sid * SL, SL)])
    plsc.subcore_barrier()

    base = wid * N_CHUNK
    p_bufs = [p0_v, p1_v]
    m_bufs = [m0_v, m1_v]
    v_bufs = [v0_v, v1_v]
    ld_sems = [ld_sem0, ld_sem1]
    st_sems = [st_sem0, st_sem1]
    loads = [None, None]
    scats = [None, None]
    loads[0] = pltpu.async_copy(pk_hbm.at[base], p_bufs[0], ld_sems[0])

    for i in range(N_CHUNK):
        b = i % 2
        nb = (i + 1) % 2
        loads[b].wait()
        if i + 1 < N_CHUNK:
            loads[nb] = pltpu.async_copy(pk_hbm.at[base + i + 1],
                                         p_bufs[nb], ld_sems[nb])
        if scats[b] is not None:
            scats[b].wait()

        pk_v, m_v, v_v = p_bufs[b], m_bufs[b], v_bufs[b]

        def unpack(g, _, pk_v=pk_v, m_v=m_v, v_v=v_v):
            off = g * (VPG * 16)
            for u in range(VPG):
                sl = pl.ds(off + u * 16, 16)
                p16 = pk_v[sl]
                z16 = lax.bitwise_and(p16, 127)
                m16 = lax.shift_right_logical(p16, 7)
                v_v[sl] = plsc.load_gather(tab_v, [z16])
                m_v[sl] = m16
            return 0

        lax.fori_loop(0, CH // (VPG * 16), unpack, 0)

        scats[b] = pltpu.async_copy(v_v, acc_sh.at[m_v],
                                    st_sems[b], add=True)

    for s in scats:
        if s is not None:
            s.wait()

    plsc.subcore_barrier()
    pltpu.sync_copy(acc_sh.at[pl.ds(sid * SL, SL)],
                    out_hbm.at[cid, pl.ds(sid * SL, SL)])


def _combine_body(mean_ref, energy_ref, n_ref, p_ref, o_ref):
    o_ref[...] = (energy_ref[...]
                  + mean_ref[0] * n_ref[...].astype(jnp.float32)
                  - p_ref[0] - p_ref[1])


def kernel(energy, n_atoms, idx_m, Z, mean, atomref):
    packed = (idx_m.astype(jnp.int32) * 128 + Z.astype(jnp.int32))
    pk = (packed.reshape(N_ATOMS // STRIDE, STRIDE).T
          .reshape(N_ATOMS // CH, CH))
    aref128 = jnp.pad(atomref.astype(jnp.float32),
                      (0, 128 - atomref.shape[0]))
    partials = _sc_scatter(pk, aref128)

    e2 = pl.pallas_call(
        _combine_body,
        out_shape=jax.ShapeDtypeStruct((128, 128), jnp.float32),
        in_specs=[
            pl.BlockSpec(memory_space=pltpu.SMEM),
            pl.BlockSpec(memory_space=pltpu.VMEM),
            pl.BlockSpec(memory_space=pltpu.VMEM),
            pl.BlockSpec(memory_space=pltpu.VMEM),
        ],
        out_specs=pl.BlockSpec(memory_space=pltpu.VMEM),
    )(mean, energy.reshape(128, 128),
      n_atoms.astype(jnp.int32).reshape(128, 128),
      partials.reshape(NC, 128, 128))
    return e2.reshape(N_MOL)
